# Initial kernel scaffold; baseline (speedup 1.0000x reference)
import jax
import jax.numpy as jnp
from jax.experimental import pallas as pl


def kernel(u_v_tensor1d, user_tensor1d, item_tensor1d, ug_u_u2, ug_v_v2, u_u_dict, user_table, item_table, uv_table, attn_W1, attn_b1, attn_W2, attn_b2, uW1, ub1, uW2, ub2, uW3, ub3, iW1, ib1, iW2, ib2, iW3, ib3, l1W, l1b, l2W, l2b, oW, ob):
    raise NotImplementedError("write your pallas kernel here")


# speedup vs baseline: 4.5087x; 4.5087x over previous
import functools

import jax
import jax.numpy as jnp
from jax import lax
from jax.experimental import pallas as pl
from jax.experimental.pallas import tpu as pltpu
from jax.experimental.pallas import tpu_sc as plsc

UN = 256
IN = 256
DE = 64
POS = UN * IN
EDGES = 4096
D1, D2 = 64, 32
DSIDE = 176

NC, NS, L = 2, 16, 16
NW = NC * NS
CHUNK = POS // NW
RB = UN // NW

_SROWS = 4
_SCOLS = POS // _SROWS


def _mm(a, b):
    return lax.dot_general(a, b, (((1,), (0,)), ((), ())),
                           preferred_element_type=jnp.float32)


def _scores_body(uv_ref, w1_ref, b1_ref, w2_ref, b2_ref, out_ref):
    x = uv_ref[...]
    h = jnp.tanh(_mm(x, w1_ref[...]) + b1_ref[...].reshape(1, 8))
    s_row = lax.dot_general(
        w2_ref[...], h, (((0,), (1,)), ((), ())),
        preferred_element_type=jnp.float32) + b2_ref[...].reshape(1, 1)
    out_ref[...] = s_row.reshape(_SCOLS)


def _scores(uv_table, w1, b1, w2, b2):
    return pl.pallas_call(
        _scores_body,
        grid=(_SROWS,),
        in_specs=[
            pl.BlockSpec((_SCOLS, DE), lambda i: (i, 0)),
            pl.BlockSpec((DE, 8), lambda i: (0, 0)),
            pl.BlockSpec((8,), lambda i: (0,)),
            pl.BlockSpec((8, 1), lambda i: (0, 0)),
            pl.BlockSpec((1,), lambda i: (0,)),
        ],
        out_specs=pl.BlockSpec((_SCOLS,), lambda i: (i,)),
        out_shape=jax.ShapeDtypeStruct((POS,), jnp.float32),
    )(uv_table, w1, b1, w2, b2)


_IDX_ROWS = CHUNK // 128


def _sc_gather_body(s_hbm, uv_idx_hbm, user_idx_hbm, item_idx_hbm,
                    uv_table_hbm, user_table_hbm, item_table_hbm,
                    w_hbm, last_social_hbm, user_emb_hbm, poi_emb_hbm,
                    idx_v, w_v, lidx_v, lrows_v, ridx_v, rows_v, sem, lsem):
    wid = lax.axis_index("s") * NC + lax.axis_index("c")
    rowbase = wid * _IDX_ROWS
    pltpu.sync_copy(uv_idx_hbm.at[pl.ds(rowbase, _IDX_ROWS)], idx_v)

    descs = [
        pltpu.async_copy(s_hbm.at[idx_v.at[j]], w_v.at[j], sem)
        for j in range(_IDX_ROWS)
    ]

    lane = lax.iota(jnp.int32, L)
    lidx_v[...] = plsc.load_gather(
        idx_v, [jnp.minimum(lane * 2 + 1, _IDX_ROWS - 1),
                jnp.full((L,), 127, jnp.int32)])
    last_desc = pltpu.async_copy(uv_table_hbm.at[lidx_v], lrows_v, lsem)

    rbase = wid * RB
    for d in descs:
        d.wait()
    pltpu.sync_copy(w_v, w_hbm.at[pl.ds(rowbase, _IDX_ROWS)])

    last_desc.wait()
    pltpu.sync_copy(lrows_v.at[pl.ds(0, RB)],
                    last_social_hbm.at[pl.ds(rbase, RB)])

    for idx_hbm, table_hbm, out_hbm in (
            (user_idx_hbm, user_table_hbm, user_emb_hbm),
            (item_idx_hbm, item_table_hbm, poi_emb_hbm)):
        pltpu.sync_copy(idx_hbm.at[pl.ds(rbase, RB)], ridx_v)
        pltpu.async_copy(table_hbm.at[ridx_v], rows_v, sem).wait()
        pltpu.sync_copy(rows_v, out_hbm.at[pl.ds(rbase, RB)])


@functools.cache
def _sc_gather_kernel():
    return pl.kernel(
        _sc_gather_body,
        out_type=[
            jax.ShapeDtypeStruct((POS // 128, 128), jnp.float32),
            jax.ShapeDtypeStruct((UN, DE), jnp.float32),
            jax.ShapeDtypeStruct((UN, DE), jnp.float32),
            jax.ShapeDtypeStruct((IN, DE), jnp.float32),
        ],
        mesh=plsc.VectorSubcoreMesh(core_axis_name="c", subcore_axis_name="s"),
        compiler_params=pltpu.CompilerParams(
            needs_layout_passes=False, use_tc_tiling_on_sc=False),
        scratch_types=[
            pltpu.VMEM((_IDX_ROWS, 128), jnp.int32),
            pltpu.VMEM((_IDX_ROWS, 128), jnp.float32),
            pltpu.VMEM((L,), jnp.int32),
            pltpu.VMEM((L, DE), jnp.float32),
            pltpu.VMEM((RB,), jnp.int32),
            pltpu.VMEM((RB, DE), jnp.float32),
            pltpu.SemaphoreType.DMA,
            pltpu.SemaphoreType.DMA,
        ],
    )


def _sc_gather(*args):
    return _sc_gather_kernel()(*args)


_TU = 256


def _fused_body(w_ref, ls_ref, ue_ref, pe_ref, ugu_ref, ugv_ref,
                uW1r, ub1r, uW2r, ub2r, uW3r, ub3r,
                iW1r, ib1r, iW2r, ib2r, iW3r, ib3r,
                l1W_ref, l1b_ref, l2w_ref, l2b_ref, ow_ref, ob_ref,
                score_ref, eu_ref, el_ref, p_s, q_s):
    step = pl.program_id(0)

    @pl.when(step == 0)
    def _prologue():
        w = w_ref[...].reshape(UN, IN)
        m = jnp.max(w, axis=1, keepdims=True)
        e = jnp.exp(w - m)
        se = jnp.sum(e, axis=1, keepdims=True)
        beta_last = e[:, IN - 1:IN] / se
        social2d = ue_ref[...] + jnp.float32(IN) * ls_ref[...] * beta_last

        eye = (lax.broadcasted_iota(jnp.int32, (UN, UN), 0) ==
               lax.broadcasted_iota(jnp.int32, (UN, UN), 1)).astype(
                   jnp.float32)

        def adjacency(edges_ref):
            src_row = edges_ref[0:1, :]
            dst_row = edges_ref[1:2, :]
            dst_t = (lax.broadcasted_iota(jnp.int32, (UN, EDGES), 0)
                     == dst_row).astype(jnp.bfloat16)
            src_t = (lax.broadcasted_iota(jnp.int32, (UN, EDGES), 0)
                     == src_row).astype(jnp.bfloat16)
            a0 = lax.dot_general(dst_t, src_t, (((1,), (1,)), ((), ())),
                                 preferred_element_type=jnp.float32)
            deg = jnp.sum(a0, axis=1, keepdims=True) + 1.0
            dinv = lax.rsqrt(deg)
            return a0 + eye, dinv

        def conv(araw, dinv, x, wr, br):
            agg = _mm(araw, dinv * _mm(x, wr[...]))
            return jnp.maximum(dinv * agg + br[...].reshape(1, -1), 0.0)

        au, dinv_u = adjacency(ugu_ref)
        eu1 = conv(au, dinv_u, social2d, uW1r, ub1r)
        eu2 = conv(au, dinv_u, eu1, uW2r, ub2r)
        eu3 = conv(au, dinv_u, eu2, uW3r, ub3r)
        av, dinv_v = adjacency(ugv_ref)
        pe = pe_ref[...]
        el1 = conv(av, dinv_v, pe, iW1r, ib1r)
        el2 = conv(av, dinv_v, el1, iW2r, ib2r)
        el3 = conv(av, dinv_v, el2, iW3r, ib3r)

        eu = jnp.concatenate([eu1, social2d, eu2, eu1, eu3, eu2], axis=-1)
        el = jnp.concatenate([el1, pe, el2, el1, el3, el2], axis=-1)
        eu_ref[...] = eu
        el_ref[...] = el
        p_s[...] = _mm(eu, l1W_ref[0:DSIDE, :])
        q_s[...] = _mm(el, l1W_ref[DSIDE:2 * DSIDE, :])

    pt = p_s[pl.ds(step * _TU, _TU), :]
    x1 = jnp.maximum(
        pt.reshape(_TU, 1, D1) + q_s[...].reshape(1, IN, D1)
        + l1b_ref[...].reshape(1, 1, D1), 0.0)
    x2 = jnp.maximum(_mm(x1.reshape(_TU * IN, D1), l2w_ref[...])
                     + l2b_ref[...].reshape(1, D2), 0.0)
    s = jnp.sum(x2.reshape(_TU, IN, D2) * ow_ref[...].reshape(1, 1, D2),
                axis=2) + ob_ref[...].reshape(1, 1)
    score_ref[...] = jnp.maximum(s, 0.0)


def _fused(w_flat, last_social, user_emb, poi_emb, ug_u, ug_v, weights):
    (uW1, ub1, uW2, ub2, uW3, ub3, iW1, ib1, iW2, ib2, iW3, ib3,
     l1W, l1b, l2W, l2b, oW, ob) = weights
    full = lambda shape: pl.BlockSpec(shape, lambda i: (0,) * len(shape))
    return pl.pallas_call(
        _fused_body,
        grid=(UN // _TU,),
        in_specs=[
            full((POS // 128, 128)), full((UN, DE)), full((UN, DE)),
            full((IN, DE)),
            full((2, EDGES)), full((2, EDGES)),
            full((DE, 32)), full((32,)), full((32, 16)), full((16,)),
            full((16, 16)), full((16,)),
            full((DE, 32)), full((32,)), full((32, 16)), full((16,)),
            full((16, 16)), full((16,)),
            full((2 * DSIDE, D1)), full((D1,)), full((D1, D2)),
            full((D2,)), full((D2, 1)), full((1,)),
        ],
        out_specs=[
            pl.BlockSpec((_TU, IN), lambda i: (i, 0)),
            full((UN, DSIDE)),
            full((IN, DSIDE)),
        ],
        out_shape=[
            jax.ShapeDtypeStruct((UN, IN), jnp.float32),
            jax.ShapeDtypeStruct((UN, DSIDE), jnp.float32),
            jax.ShapeDtypeStruct((IN, DSIDE), jnp.float32),
        ],
        scratch_shapes=[
            pltpu.VMEM((UN, D1), jnp.float32),
            pltpu.VMEM((IN, D1), jnp.float32),
        ],
    )(w_flat, last_social, user_emb, poi_emb, ug_u, ug_v,
      uW1, ub1, uW2, ub2, uW3, ub3,
      iW1, ib1, iW2, ib2, iW3, ib3,
      l1W, l1b, l2W, l2b, oW, ob)


def kernel(u_v_tensor1d, user_tensor1d, item_tensor1d, ug_u_u2, ug_v_v2,
           u_u_dict, user_table, item_table, uv_table, attn_W1, attn_b1,
           attn_W2, attn_b2, uW1, ub1, uW2, ub2, uW3, ub3, iW1, ib1, iW2,
           ib2, iW3, ib3, l1W, l1b, l2W, l2b, oW, ob):
    s = _scores(uv_table, attn_W1, attn_b1, attn_W2, attn_b2)
    w_flat, last_social, user_emb, poi_emb = _sc_gather(
        s, u_v_tensor1d.reshape(POS // 128, 128), user_tensor1d,
        item_tensor1d, uv_table, user_table, item_table)
    score, eu, el = _fused(
        w_flat, last_social, user_emb, poi_emb,
        ug_u_u2, ug_v_v2,
        (uW1, ub1, uW2, ub2, uW3, ub3, iW1, ib1, iW2, ib2, iW3, ib3,
         l1W, l1b, l2W, l2b, oW, ob))
    return score, eu, el


# speedup vs baseline: 4.5742x; 1.0145x over previous
import functools

import jax
import jax.numpy as jnp
from jax import lax
from jax.experimental import pallas as pl
from jax.experimental.pallas import tpu as pltpu
from jax.experimental.pallas import tpu_sc as plsc

UN = 256
IN = 256
DE = 64
POS = UN * IN
EDGES = 4096
D1, D2 = 64, 32
DSIDE = 176

NC, NS, L = 2, 16, 16
NW = NC * NS
CHUNK = POS // NW
RB = UN // NW

_SROWS = 4
_SCOLS = POS // _SROWS


def _mm(a, b):
    return lax.dot_general(a, b, (((1,), (0,)), ((), ())),
                           preferred_element_type=jnp.float32)


def _scores_body(uv_ref, w1_ref, b1_ref, w2_ref, b2_ref, out_ref):
    x = uv_ref[...]
    h = jnp.tanh(_mm(x, w1_ref[...]) + b1_ref[...].reshape(1, 8))
    s_row = lax.dot_general(
        w2_ref[...], h, (((0,), (1,)), ((), ())),
        preferred_element_type=jnp.float32) + b2_ref[...].reshape(1, 1)
    out_ref[...] = s_row.reshape(_SCOLS)


def _scores(uv_table, w1, b1, w2, b2):
    return pl.pallas_call(
        _scores_body,
        grid=(_SROWS,),
        in_specs=[
            pl.BlockSpec((_SCOLS, DE), lambda i: (i, 0)),
            pl.BlockSpec((DE, 8), lambda i: (0, 0)),
            pl.BlockSpec((8,), lambda i: (0,)),
            pl.BlockSpec((8, 1), lambda i: (0, 0)),
            pl.BlockSpec((1,), lambda i: (0,)),
        ],
        out_specs=pl.BlockSpec((_SCOLS,), lambda i: (i,)),
        out_shape=jax.ShapeDtypeStruct((POS,), jnp.float32),
    )(uv_table, w1, b1, w2, b2)


_IDX_ROWS = CHUNK // 128


def _sc_rows_body(uv_idx_hbm, user_idx_hbm, item_idx_hbm,
                  uv_table_hbm, user_table_hbm, item_table_hbm,
                  last_social_hbm, user_emb_hbm, poi_emb_hbm,
                  idx_v, lidx_v, lrows_v, ridx_v, rows_v, sem, lsem):
    wid = lax.axis_index("s") * NC + lax.axis_index("c")
    rowbase = wid * _IDX_ROWS
    pltpu.sync_copy(uv_idx_hbm.at[pl.ds(rowbase, _IDX_ROWS)], idx_v)

    lane = lax.iota(jnp.int32, L)
    lidx_v[...] = plsc.load_gather(
        idx_v, [jnp.minimum(lane * 2 + 1, _IDX_ROWS - 1),
                jnp.full((L,), 127, jnp.int32)])
    last_desc = pltpu.async_copy(uv_table_hbm.at[lidx_v], lrows_v, lsem)

    rbase = wid * RB
    last_desc.wait()
    pltpu.sync_copy(lrows_v.at[pl.ds(0, RB)],
                    last_social_hbm.at[pl.ds(rbase, RB)])

    for idx_hbm, table_hbm, out_hbm in (
            (user_idx_hbm, user_table_hbm, user_emb_hbm),
            (item_idx_hbm, item_table_hbm, poi_emb_hbm)):
        pltpu.sync_copy(idx_hbm.at[pl.ds(rbase, RB)], ridx_v)
        pltpu.async_copy(table_hbm.at[ridx_v], rows_v, sem).wait()
        pltpu.sync_copy(rows_v, out_hbm.at[pl.ds(rbase, RB)])


def _sc_logits_body(s_hbm, uv_idx_hbm, w_hbm, idx_v, w_v, sem):
    wid = lax.axis_index("s") * NC + lax.axis_index("c")
    rowbase = wid * _IDX_ROWS
    pltpu.sync_copy(uv_idx_hbm.at[pl.ds(rowbase, _IDX_ROWS)], idx_v)

    descs = [
        pltpu.async_copy(s_hbm.at[idx_v.at[j]], w_v.at[j], sem)
        for j in range(_IDX_ROWS)
    ]
    for d in descs:
        d.wait()
    pltpu.sync_copy(w_v, w_hbm.at[pl.ds(rowbase, _IDX_ROWS)])


@functools.cache
def _sc_rows_kernel():
    return pl.kernel(
        _sc_rows_body,
        out_type=[
            jax.ShapeDtypeStruct((UN, DE), jnp.float32),
            jax.ShapeDtypeStruct((UN, DE), jnp.float32),
            jax.ShapeDtypeStruct((IN, DE), jnp.float32),
        ],
        mesh=plsc.VectorSubcoreMesh(core_axis_name="c", subcore_axis_name="s"),
        compiler_params=pltpu.CompilerParams(
            needs_layout_passes=False, use_tc_tiling_on_sc=False),
        scratch_types=[
            pltpu.VMEM((_IDX_ROWS, 128), jnp.int32),
            pltpu.VMEM((L,), jnp.int32),
            pltpu.VMEM((L, DE), jnp.float32),
            pltpu.VMEM((RB,), jnp.int32),
            pltpu.VMEM((RB, DE), jnp.float32),
            pltpu.SemaphoreType.DMA,
            pltpu.SemaphoreType.DMA,
        ],
    )


@functools.cache
def _sc_logits_kernel():
    return pl.kernel(
        _sc_logits_body,
        out_type=[
            jax.ShapeDtypeStruct((POS // 128, 128), jnp.float32),
        ],
        mesh=plsc.VectorSubcoreMesh(core_axis_name="c", subcore_axis_name="s"),
        compiler_params=pltpu.CompilerParams(
            needs_layout_passes=False, use_tc_tiling_on_sc=False),
        scratch_types=[
            pltpu.VMEM((_IDX_ROWS, 128), jnp.int32),
            pltpu.VMEM((_IDX_ROWS, 128), jnp.float32),
            pltpu.SemaphoreType.DMA,
        ],
    )


_TU = 256


def _fused_body(w_ref, ls_ref, ue_ref, pe_ref, ugu_ref, ugv_ref,
                uW1r, ub1r, uW2r, ub2r, uW3r, ub3r,
                iW1r, ib1r, iW2r, ib2r, iW3r, ib3r,
                l1W_ref, l1b_ref, l2w_ref, l2b_ref, ow_ref, ob_ref,
                score_ref, eu_ref, el_ref, p_s, q_s):
    step = pl.program_id(0)

    @pl.when(step == 0)
    def _prologue():
        w = w_ref[...].reshape(UN, IN)
        m = jnp.max(w, axis=1, keepdims=True)
        e = jnp.exp(w - m)
        se = jnp.sum(e, axis=1, keepdims=True)
        beta_last = e[:, IN - 1:IN] / se
        social2d = ue_ref[...] + jnp.float32(IN) * ls_ref[...] * beta_last

        eye = (lax.broadcasted_iota(jnp.int32, (UN, UN), 0) ==
               lax.broadcasted_iota(jnp.int32, (UN, UN), 1)).astype(
                   jnp.float32)

        def adjacency(edges_ref):
            src_row = edges_ref[0:1, :]
            dst_row = edges_ref[1:2, :]
            dst_t = (lax.broadcasted_iota(jnp.int32, (UN, EDGES), 0)
                     == dst_row).astype(jnp.bfloat16)
            src_t = (lax.broadcasted_iota(jnp.int32, (UN, EDGES), 0)
                     == src_row).astype(jnp.bfloat16)
            a0 = lax.dot_general(dst_t, src_t, (((1,), (1,)), ((), ())),
                                 preferred_element_type=jnp.float32)
            deg = jnp.sum(a0, axis=1, keepdims=True) + 1.0
            dinv = lax.rsqrt(deg)
            return a0 + eye, dinv

        def conv(araw, dinv, x, wr, br):
            agg = _mm(araw, dinv * _mm(x, wr[...]))
            return jnp.maximum(dinv * agg + br[...].reshape(1, -1), 0.0)

        au, dinv_u = adjacency(ugu_ref)
        eu1 = conv(au, dinv_u, social2d, uW1r, ub1r)
        eu2 = conv(au, dinv_u, eu1, uW2r, ub2r)
        eu3 = conv(au, dinv_u, eu2, uW3r, ub3r)
        av, dinv_v = adjacency(ugv_ref)
        pe = pe_ref[...]
        el1 = conv(av, dinv_v, pe, iW1r, ib1r)
        el2 = conv(av, dinv_v, el1, iW2r, ib2r)
        el3 = conv(av, dinv_v, el2, iW3r, ib3r)

        eu = jnp.concatenate([eu1, social2d, eu2, eu1, eu3, eu2], axis=-1)
        el = jnp.concatenate([el1, pe, el2, el1, el3, el2], axis=-1)
        eu_ref[...] = eu
        el_ref[...] = el
        p_s[...] = _mm(eu, l1W_ref[0:DSIDE, :])
        q_s[...] = _mm(el, l1W_ref[DSIDE:2 * DSIDE, :])

    pt = p_s[pl.ds(step * _TU, _TU), :]
    x1 = jnp.maximum(
        pt.reshape(_TU, 1, D1) + q_s[...].reshape(1, IN, D1)
        + l1b_ref[...].reshape(1, 1, D1), 0.0)
    x2 = jnp.maximum(_mm(x1.reshape(_TU * IN, D1), l2w_ref[...])
                     + l2b_ref[...].reshape(1, D2), 0.0)
    s = jnp.sum(x2.reshape(_TU, IN, D2) * ow_ref[...].reshape(1, 1, D2),
                axis=2) + ob_ref[...].reshape(1, 1)
    score_ref[...] = jnp.maximum(s, 0.0)


def _fused(w_flat, last_social, user_emb, poi_emb, ug_u, ug_v, weights):
    (uW1, ub1, uW2, ub2, uW3, ub3, iW1, ib1, iW2, ib2, iW3, ib3,
     l1W, l1b, l2W, l2b, oW, ob) = weights
    full = lambda shape: pl.BlockSpec(shape, lambda i: (0,) * len(shape))
    return pl.pallas_call(
        _fused_body,
        grid=(UN // _TU,),
        in_specs=[
            full((POS // 128, 128)), full((UN, DE)), full((UN, DE)),
            full((IN, DE)),
            full((2, EDGES)), full((2, EDGES)),
            full((DE, 32)), full((32,)), full((32, 16)), full((16,)),
            full((16, 16)), full((16,)),
            full((DE, 32)), full((32,)), full((32, 16)), full((16,)),
            full((16, 16)), full((16,)),
            full((2 * DSIDE, D1)), full((D1,)), full((D1, D2)),
            full((D2,)), full((D2, 1)), full((1,)),
        ],
        out_specs=[
            pl.BlockSpec((_TU, IN), lambda i: (i, 0)),
            full((UN, DSIDE)),
            full((IN, DSIDE)),
        ],
        out_shape=[
            jax.ShapeDtypeStruct((UN, IN), jnp.float32),
            jax.ShapeDtypeStruct((UN, DSIDE), jnp.float32),
            jax.ShapeDtypeStruct((IN, DSIDE), jnp.float32),
        ],
        scratch_shapes=[
            pltpu.VMEM((UN, D1), jnp.float32),
            pltpu.VMEM((IN, D1), jnp.float32),
        ],
    )(w_flat, last_social, user_emb, poi_emb, ug_u, ug_v,
      uW1, ub1, uW2, ub2, uW3, ub3,
      iW1, ib1, iW2, ib2, iW3, ib3,
      l1W, l1b, l2W, l2b, oW, ob)


def kernel(u_v_tensor1d, user_tensor1d, item_tensor1d, ug_u_u2, ug_v_v2,
           u_u_dict, user_table, item_table, uv_table, attn_W1, attn_b1,
           attn_W2, attn_b2, uW1, ub1, uW2, ub2, uW3, ub3, iW1, ib1, iW2,
           ib2, iW3, ib3, l1W, l1b, l2W, l2b, oW, ob):
    uv_idx = u_v_tensor1d.reshape(POS // 128, 128)
    last_social, user_emb, poi_emb = _sc_rows_kernel()(
        uv_idx, user_tensor1d, item_tensor1d,
        uv_table, user_table, item_table)
    s = _scores(uv_table, attn_W1, attn_b1, attn_W2, attn_b2)
    (w_flat,) = _sc_logits_kernel()(s, uv_idx)
    score, eu, el = _fused(
        w_flat, last_social, user_emb, poi_emb,
        ug_u_u2, ug_v_v2,
        (uW1, ub1, uW2, ub2, uW3, ub3, iW1, ib1, iW2, ib2, iW3, ib3,
         l1W, l1b, l2W, l2b, oW, ob))
    return score, eu, el
